# Initial kernel scaffold; baseline (speedup 1.0000x reference)
import jax
import jax.numpy as jnp
from jax.experimental import pallas as pl


def kernel(x_s, edge_index_s, x_t, edge_index_t, W1, b1, W2, b2, W3, b3, Wl, bl):
    raise NotImplementedError("write your pallas kernel here")


# speedup vs baseline: 9.5337x; 9.5337x over previous
import functools

import jax
import jax.numpy as jnp
from jax import lax
from jax.experimental import pallas as pl
from jax.experimental.pallas import tpu as pltpu
from jax.experimental.pallas import tpu_sc as plsc

N = 10000
E = 320000
NP = 10240
NS = 16
CH = 128
CPT = 160
SUB = 4
CPS = CPT // SUB
EPT = CPT * CH
EP = NS * EPT
RPT = NP // NS
D1 = 128
D3 = 16
RB = 1024

_mesh = plsc.VectorSubcoreMesh(core_axis_name="c", subcore_axis_name="s")


def _agg_body(z_hbm, rows_hbm, cols_hbm, zeros_hbm, out_hbm,
              rowv, colv, buf0, buf1, acc, sem0, sem1):
    c = lax.axis_index("c")
    s = lax.axis_index("s")
    w = (c * NS + s) * SUB
    pltpu.sync_copy(zeros_hbm, acc.at[pl.ds(s * RPT, RPT)])
    plsc.subcore_barrier()

    def step(j, carry):
        jj = j * 2
        g0 = pltpu.async_copy(z_hbm.at[rowv.at[jj]], buf0, sem0)
        g1 = pltpu.async_copy(z_hbm.at[rowv.at[jj + 1]], buf1, sem1)
        g0.wait()
        pltpu.sync_copy(buf0, acc.at[colv.at[jj]], add=True)
        g1.wait()
        pltpu.sync_copy(buf1, acc.at[colv.at[jj + 1]], add=True)
        return carry

    for u in range(SUB):
        pltpu.sync_copy(rows_hbm.at[w + u], rowv)
        pltpu.sync_copy(cols_hbm.at[w + u], colv)
        lax.fori_loop(0, CPS // 2, step, 0)
    plsc.subcore_barrier()
    pltpu.sync_copy(acc.at[pl.ds(s * RPT, RPT)],
                    out_hbm.at[pl.ds(c * NP + s * RPT, RPT)])


_agg128 = pl.kernel(
    _agg_body,
    out_type=jax.ShapeDtypeStruct((2 * NP, D1), jnp.float32),
    mesh=_mesh,
    scratch_types=[
        pltpu.VMEM((CPS, CH), jnp.int32),
        pltpu.VMEM((CPS, CH), jnp.int32),
        pltpu.VMEM((CH, D1), jnp.float32),
        pltpu.VMEM((CH, D1), jnp.float32),
        pltpu.VMEM_SHARED((NP, D1), jnp.float32),
        pltpu.SemaphoreType.DMA,
        pltpu.SemaphoreType.DMA,
    ],
)


def _agg16_body(z_hbm, rows_hbm, cols_hbm, zeros_hbm, out_hbm,
                rowv, colv, buf0, buf1, zsh, acc, sem0, sem1):
    c = lax.axis_index("c")
    s = lax.axis_index("s")
    w = c * NS + s
    pltpu.sync_copy(rows_hbm.at[w], rowv)
    pltpu.sync_copy(cols_hbm.at[w], colv)
    pltpu.sync_copy(z_hbm.at[pl.ds(c * NP + s * RPT, RPT)],
                    zsh.at[pl.ds(s * RPT, RPT)])
    pltpu.sync_copy(zeros_hbm, acc.at[pl.ds(s * RPT, RPT)])
    plsc.subcore_barrier()

    def step(j, carry):
        jj = j * 2
        g0 = pltpu.async_copy(zsh.at[rowv.at[jj]], buf0, sem0)
        g1 = pltpu.async_copy(zsh.at[rowv.at[jj + 1]], buf1, sem1)
        g0.wait()
        pltpu.sync_copy(buf0, acc.at[colv.at[jj]], add=True)
        g1.wait()
        pltpu.sync_copy(buf1, acc.at[colv.at[jj + 1]], add=True)
        return carry

    lax.fori_loop(0, CPT // 2, step, 0)
    plsc.subcore_barrier()
    pltpu.sync_copy(acc.at[pl.ds(s * RPT, RPT)],
                    out_hbm.at[pl.ds(c * NP + s * RPT, RPT)])


_agg16 = pl.kernel(
    _agg16_body,
    out_type=jax.ShapeDtypeStruct((2 * NP, D3), jnp.float32),
    mesh=_mesh,
    scratch_types=[
        pltpu.VMEM((CPT, CH), jnp.int32),
        pltpu.VMEM((CPT, CH), jnp.int32),
        pltpu.VMEM((CH, D3), jnp.float32),
        pltpu.VMEM((CH, D3), jnp.float32),
        pltpu.VMEM_SHARED((NP, D3), jnp.float32),
        pltpu.VMEM_SHARED((NP, D3), jnp.float32),
        pltpu.SemaphoreType.DMA,
        pltpu.SemaphoreType.DMA,
    ],
)


def _kw_body(a_ref, w2_ref, out_ref):
    out_ref[...] = jnp.dot(a_ref[...], w2_ref[...],
                           preferred_element_type=jnp.float32)


def _k1_body(cnt_ref, msk_ref, x_ref, dinv_ref, z0_ref):
    dinv = msk_ref[...] * lax.rsqrt(1.0 + cnt_ref[...][:, :1])
    dinvb = jnp.broadcast_to(dinv, (RB, D1))
    dinv_ref[...] = dinvb
    z0_ref[...] = dinvb * x_ref[...]


def _k2_body(acc_ref, z0_ref, dinv_ref, w12_ref, b12_ref, z2_ref):
    dinv = dinv_ref[...]
    a1 = dinv * (acc_ref[...] + z0_ref[...])
    h2 = jnp.dot(a1, w12_ref[...], preferred_element_type=jnp.float32)
    z2_ref[...] = dinv * (h2 + b12_ref[...])


def _k3_body(acc_ref, z2_ref, dinv_ref, w3_ref, b2_ref, z3_ref):
    dinv = dinv_ref[...]
    x2 = dinv * (acc_ref[...] + z2_ref[...]) + b2_ref[...]
    h3 = jnp.dot(x2, w3_ref[...], preferred_element_type=jnp.float32)
    z3 = dinv[:, :D3] * h3
    z3_ref[...] = jnp.concatenate(
        [z3, jnp.zeros((RB, D1 - D3), jnp.float32)], axis=1)


def _k4_body(acc_ref, z3_ref, dinv_ref, b3_ref, wl_ref, bl_ref,
             x3_ref, p_ref):
    x3 = (dinv_ref[...][:, :D3] * (acc_ref[...][:, :D3] + z3_ref[...][:, :D3])
          + b3_ref[...])
    x3_ref[...] = x3
    logits = jnp.dot(x3, wl_ref[...],
                     preferred_element_type=jnp.float32) + bl_ref[...]
    p_ref[...] = jax.nn.sigmoid(logits)


def _row_spec(w):
    return pl.BlockSpec((RB, w), lambda i: (i, 0))


def _full_spec(shape):
    return pl.BlockSpec(shape, lambda i: tuple(0 for _ in shape))


_GRID = (2 * NP // RB,)

_kw = pl.pallas_call(
    _kw_body,
    grid=(1,),
    in_specs=[_full_spec((136, 1000)), _full_spec((1000, D1))],
    out_specs=_full_spec((136, D1)),
    out_shape=jax.ShapeDtypeStruct((136, D1), jnp.float32),
)

_k1 = pl.pallas_call(
    _k1_body,
    grid=_GRID,
    in_specs=[_row_spec(D1), _row_spec(1), _row_spec(D1)],
    out_specs=[_row_spec(D1), _row_spec(D1)],
    out_shape=[jax.ShapeDtypeStruct((2 * NP, D1), jnp.float32),
               jax.ShapeDtypeStruct((2 * NP, D1), jnp.float32)],
)

_k2 = pl.pallas_call(
    _k2_body,
    grid=_GRID,
    in_specs=[_row_spec(D1), _row_spec(D1), _row_spec(D1),
              _full_spec((D1, D1)), _full_spec((1, D1))],
    out_specs=_row_spec(D1),
    out_shape=jax.ShapeDtypeStruct((2 * NP, D1), jnp.float32),
)

_k3 = pl.pallas_call(
    _k3_body,
    grid=_GRID,
    in_specs=[_row_spec(D1), _row_spec(D1), _row_spec(D1),
              _full_spec((D1, D3)), _full_spec((1, D1))],
    out_specs=_row_spec(D1),
    out_shape=jax.ShapeDtypeStruct((2 * NP, D1), jnp.float32),
)

_k4 = pl.pallas_call(
    _k4_body,
    grid=_GRID,
    in_specs=[_row_spec(D1), _row_spec(D1), _row_spec(D1),
              _full_spec((1, D3)), _full_spec((D3, D3)), _full_spec((1, D3))],
    out_specs=[_row_spec(D3), _row_spec(D3)],
    out_shape=[jax.ShapeDtypeStruct((2 * NP, D3), jnp.float32),
               jax.ShapeDtypeStruct((2 * NP, D3), jnp.float32)],
)


def _prep_edges(ei, g):
    pad = jnp.full((EP - E,), N, jnp.int32)
    rl = jnp.concatenate([ei[0], pad])
    c = jnp.concatenate([ei[1], pad])
    rg = rl + jnp.int32(g * NP)
    return (rg.reshape(NS * SUB, CPS, CH), c.reshape(NS * SUB, CPS, CH),
            rl.reshape(NS, CPT, CH), c.reshape(NS, CPT, CH))


def kernel(x_s, edge_index_s, x_t, edge_index_t,
           W1, b1, W2, b2, W3, b3, Wl, bl):
    f32 = jnp.float32
    rs, cs, rls, cls = _prep_edges(edge_index_s, 0)
    rt, ct, rlt, clt = _prep_edges(edge_index_t, 1)
    rows = jnp.concatenate([rs, rt], axis=0)
    cols = jnp.concatenate([cs, ct], axis=0)
    rowsl = jnp.concatenate([rls, rlt], axis=0)
    colsl = jnp.concatenate([cls, clt], axis=0)

    padrows = jnp.zeros((NP - N, x_s.shape[1]), f32)
    x2g = jnp.concatenate([x_s, padrows, x_t, padrows], axis=0)

    mask = jnp.concatenate([jnp.ones((N, 1), f32), jnp.zeros((NP - N, 1), f32)])
    mask = jnp.concatenate([mask, mask], axis=0)

    a_in = jnp.concatenate([W1, b1[None, :], jnp.zeros((7, 1000), f32)])
    w2p = jnp.pad(W2, ((0, 0), (0, D1 - 100)))
    wb12 = _kw(a_in, w2p)
    w12, b12 = wb12[:128], wb12[128:129]
    w3p = jnp.pad(W3, ((0, D1 - 100), (0, 0)))
    b2p = jnp.pad(b2, (0, D1 - 100))[None, :]
    wlp = jnp.pad(Wl, ((0, 0), (0, D3 - 10)))
    blp = jnp.pad(bl, (0, D3 - 10))[None, :]

    z128 = jnp.zeros((RPT, D1), f32)
    z16 = jnp.zeros((RPT, D3), f32)

    cnt = _agg128(jnp.ones((2 * NP, D1), f32), rows, cols, z128)
    dinvb, z0 = _k1(cnt, mask, x2g)
    acc1 = _agg128(z0, rows, cols, z128)
    z2 = _k2(acc1, z0, dinvb, w12, b12)
    acc2 = _agg128(z2, rows, cols, z128)
    z3 = _k3(acc2, z2, dinvb, w3p, b2p)
    acc3 = _agg128(z3, rows, cols, z128)
    x3, p = _k4(acc3, z3, dinvb, b3[None, :], wlp, blp)

    x_s_out = x3[:N]
    x_t_out = x3[NP:NP + N]
    pred = p[:N, :10]
    return (x_s_out, x_t_out, pred, pred)


# speedup vs baseline: 11.3121x; 1.1865x over previous
import jax
import jax.numpy as jnp
from jax import lax
from jax.experimental import pallas as pl
from jax.experimental.pallas import tpu as pltpu
from jax.experimental.pallas import tpu_sc as plsc

N = 10000
E = 320000
NP = 10240
NS = 16
CH = 64
NBUF = 4
CPT = 320
SUB = 8
CPS = CPT // SUB
EPT = CPT * CH
EP = NS * EPT
RPT = NP // NS
D1 = 128
D3 = 16
RB = 1024

_mesh = plsc.VectorSubcoreMesh(core_axis_name="c", subcore_axis_name="s")


def _agg_body(z_hbm, rows_hbm, cols_hbm, zeros_hbm, out_hbm,
              rowv, colv, buf0, buf1, buf2, buf3, acc,
              gsem0, gsem1, gsem2, gsem3, ssem0, ssem1, ssem2, ssem3):
    bufs = (buf0, buf1, buf2, buf3)
    gsems = (gsem0, gsem1, gsem2, gsem3)
    ssems = (ssem0, ssem1, ssem2, ssem3)
    c = lax.axis_index("c")
    s = lax.axis_index("s")
    w = (c * NS + s) * SUB
    pltpu.sync_copy(zeros_hbm, acc.at[pl.ds(s * RPT, RPT)])
    plsc.subcore_barrier()

    for u in range(SUB):
        pltpu.sync_copy(rows_hbm.at[w + u], rowv)
        pltpu.sync_copy(cols_hbm.at[w + u], colv)
        for b in range(NBUF):
            pltpu.async_copy(z_hbm.at[rowv.at[b]], bufs[b], gsems[b])

        def round_fn(r, carry):
            sdescs = []
            for b in range(NBUF):
                j = r * NBUF + b
                pltpu.make_async_copy(
                    z_hbm.at[rowv.at[j]], bufs[b], gsems[b]).wait()
                sdescs.append(pltpu.async_copy(
                    bufs[b], acc.at[colv.at[j]], ssems[b], add=True))
            for b in range(NBUF):
                sdescs[b].wait()
                pltpu.async_copy(z_hbm.at[rowv.at[(r + 1) * NBUF + b]],
                                 bufs[b], gsems[b])
            return carry

        lax.fori_loop(0, CPS // NBUF - 1, round_fn, 0)
        for b in range(NBUF):
            j = CPS - NBUF + b
            pltpu.make_async_copy(
                z_hbm.at[rowv.at[j]], bufs[b], gsems[b]).wait()
            pltpu.sync_copy(bufs[b], acc.at[colv.at[j]], add=True)
    plsc.subcore_barrier()
    pltpu.sync_copy(acc.at[pl.ds(s * RPT, RPT)],
                    out_hbm.at[pl.ds(c * NP + s * RPT, RPT)])


_agg128 = pl.kernel(
    _agg_body,
    out_type=jax.ShapeDtypeStruct((2 * NP, D1), jnp.float32),
    mesh=_mesh,
    scratch_types=(
        [pltpu.VMEM((CPS, CH), jnp.int32),
         pltpu.VMEM((CPS, CH), jnp.int32)]
        + [pltpu.VMEM((CH, D1), jnp.float32) for _ in range(NBUF)]
        + [pltpu.VMEM_SHARED((NP, D1), jnp.float32)]
        + [pltpu.SemaphoreType.DMA for _ in range(2 * NBUF)]
    ),
)


def _kw_body(a_ref, w2_ref, out_ref):
    out_ref[...] = jnp.dot(a_ref[...], w2_ref[...],
                           preferred_element_type=jnp.float32)


def _k1_body(cnt_ref, msk_ref, x_ref, dinv_ref, z0_ref):
    dinv = msk_ref[...] * lax.rsqrt(1.0 + cnt_ref[...][:, :1])
    dinvb = jnp.broadcast_to(dinv, (RB, D1))
    dinv_ref[...] = dinvb
    z0_ref[...] = dinvb * x_ref[...]


def _k2_body(acc_ref, z0_ref, dinv_ref, w12_ref, b12_ref, z2_ref):
    dinv = dinv_ref[...]
    a1 = dinv * (acc_ref[...] + z0_ref[...])
    h2 = jnp.dot(a1, w12_ref[...], preferred_element_type=jnp.float32)
    z2_ref[...] = dinv * (h2 + b12_ref[...])


def _k3_body(acc_ref, z2_ref, dinv_ref, w3_ref, b2_ref, z3_ref):
    dinv = dinv_ref[...]
    x2 = dinv * (acc_ref[...] + z2_ref[...]) + b2_ref[...]
    h3 = jnp.dot(x2, w3_ref[...], preferred_element_type=jnp.float32)
    z3 = dinv[:, :D3] * h3
    z3_ref[...] = jnp.concatenate(
        [z3, jnp.zeros((RB, D1 - D3), jnp.float32)], axis=1)


def _k4_body(acc_ref, z3_ref, dinv_ref, b3_ref, wl_ref, bl_ref,
             x3_ref, p_ref):
    x3 = (dinv_ref[...][:, :D3] * (acc_ref[...][:, :D3] + z3_ref[...][:, :D3])
          + b3_ref[...])
    x3_ref[...] = x3
    logits = jnp.dot(x3, wl_ref[...],
                     preferred_element_type=jnp.float32) + bl_ref[...]
    p_ref[...] = jax.nn.sigmoid(logits)


def _row_spec(w):
    return pl.BlockSpec((RB, w), lambda i: (i, 0))


def _full_spec(shape):
    return pl.BlockSpec(shape, lambda i: tuple(0 for _ in shape))


_GRID = (2 * NP // RB,)

_kw = pl.pallas_call(
    _kw_body,
    grid=(1,),
    in_specs=[_full_spec((136, 1000)), _full_spec((1000, D1))],
    out_specs=_full_spec((136, D1)),
    out_shape=jax.ShapeDtypeStruct((136, D1), jnp.float32),
)

_k1 = pl.pallas_call(
    _k1_body,
    grid=_GRID,
    in_specs=[_row_spec(D1), _row_spec(1), _row_spec(D1)],
    out_specs=[_row_spec(D1), _row_spec(D1)],
    out_shape=[jax.ShapeDtypeStruct((2 * NP, D1), jnp.float32),
               jax.ShapeDtypeStruct((2 * NP, D1), jnp.float32)],
)

_k2 = pl.pallas_call(
    _k2_body,
    grid=_GRID,
    in_specs=[_row_spec(D1), _row_spec(D1), _row_spec(D1),
              _full_spec((D1, D1)), _full_spec((1, D1))],
    out_specs=_row_spec(D1),
    out_shape=jax.ShapeDtypeStruct((2 * NP, D1), jnp.float32),
)

_k3 = pl.pallas_call(
    _k3_body,
    grid=_GRID,
    in_specs=[_row_spec(D1), _row_spec(D1), _row_spec(D1),
              _full_spec((D1, D3)), _full_spec((1, D1))],
    out_specs=_row_spec(D1),
    out_shape=jax.ShapeDtypeStruct((2 * NP, D1), jnp.float32),
)

_k4 = pl.pallas_call(
    _k4_body,
    grid=_GRID,
    in_specs=[_row_spec(D1), _row_spec(D1), _row_spec(D1),
              _full_spec((1, D3)), _full_spec((D3, D3)), _full_spec((1, D3))],
    out_specs=[_row_spec(D3), _row_spec(D3)],
    out_shape=[jax.ShapeDtypeStruct((2 * NP, D3), jnp.float32),
               jax.ShapeDtypeStruct((2 * NP, D3), jnp.float32)],
)


def _prep_edges(ei, g):
    pad = jnp.full((EP - E,), N, jnp.int32)
    rl = jnp.concatenate([ei[0], pad])
    c = jnp.concatenate([ei[1], pad])
    rg = rl + jnp.int32(g * NP)
    return rg.reshape(NS * SUB, CPS, CH), c.reshape(NS * SUB, CPS, CH)


def kernel(x_s, edge_index_s, x_t, edge_index_t,
           W1, b1, W2, b2, W3, b3, Wl, bl):
    f32 = jnp.float32
    rs, cs = _prep_edges(edge_index_s, 0)
    rt, ct = _prep_edges(edge_index_t, 1)
    rows = jnp.concatenate([rs, rt], axis=0)
    cols = jnp.concatenate([cs, ct], axis=0)

    padrows = jnp.zeros((NP - N, x_s.shape[1]), f32)
    x2g = jnp.concatenate([x_s, padrows, x_t, padrows], axis=0)

    mask = jnp.concatenate([jnp.ones((N, 1), f32), jnp.zeros((NP - N, 1), f32)])
    mask = jnp.concatenate([mask, mask], axis=0)

    a_in = jnp.concatenate([W1, b1[None, :], jnp.zeros((7, 1000), f32)])
    w2p = jnp.pad(W2, ((0, 0), (0, D1 - 100)))
    wb12 = _kw(a_in, w2p)
    w12, b12 = wb12[:128], wb12[128:129]
    w3p = jnp.pad(W3, ((0, D1 - 100), (0, 0)))
    b2p = jnp.pad(b2, (0, D1 - 100))[None, :]
    wlp = jnp.pad(Wl, ((0, 0), (0, D3 - 10)))
    blp = jnp.pad(bl, (0, D3 - 10))[None, :]

    z128 = jnp.zeros((RPT, D1), f32)
    z16 = jnp.zeros((RPT, D3), f32)

    cnt = _agg128(jnp.ones((2 * NP, D1), f32), rows, cols, z128)
    dinvb, z0 = _k1(cnt, mask, x2g)
    acc1 = _agg128(z0, rows, cols, z128)
    z2 = _k2(acc1, z0, dinvb, w12, b12)
    acc2 = _agg128(z2, rows, cols, z128)
    z3 = _k3(acc2, z2, dinvb, w3p, b2p)
    acc3 = _agg128(z3, rows, cols, z128)
    x3, p = _k4(acc3, z3, dinvb, b3[None, :], wlp, blp)

    x_s_out = x3[:N]
    x_t_out = x3[NP:NP + N]
    pred = p[:N, :10]
    return (x_s_out, x_t_out, pred, pred)


# speedup vs baseline: 12.3069x; 1.0879x over previous
import jax
import jax.numpy as jnp
from jax import lax
from jax.experimental import pallas as pl
from jax.experimental.pallas import tpu as pltpu
from jax.experimental.pallas import tpu_sc as plsc

N = 10000
E = 320000
NP = 10240
NS = 16
CH = 64
NBUF = 4
CPT = 320
SUB = 8
CPS = CPT // SUB
EPT = CPT * CH
EP = NS * EPT
RPT = NP // NS
D1 = 128
D3 = 16
RB = 1024

_mesh = plsc.VectorSubcoreMesh(core_axis_name="c", subcore_axis_name="s")


def _agg_body(z_hbm, rows_hbm, cols_hbm, zeros_hbm, out_hbm,
              rowv, colv, buf0, buf1, buf2, buf3, acc,
              gsem0, gsem1, gsem2, gsem3, ssem0, ssem1, ssem2, ssem3):
    bufs = (buf0, buf1, buf2, buf3)
    gsems = (gsem0, gsem1, gsem2, gsem3)
    ssems = (ssem0, ssem1, ssem2, ssem3)
    c = lax.axis_index("c")
    s = lax.axis_index("s")
    w = (c * NS + s) * SUB
    pltpu.sync_copy(zeros_hbm, acc.at[pl.ds(s * RPT, RPT)])
    plsc.subcore_barrier()

    for u in range(SUB):
        pltpu.sync_copy(rows_hbm.at[w + u], rowv)
        pltpu.sync_copy(cols_hbm.at[w + u], colv)
        for b in range(NBUF):
            pltpu.async_copy(z_hbm.at[rowv.at[b]], bufs[b], gsems[b])

        def round_fn(r, carry):
            sdescs = []
            for b in range(NBUF):
                j = r * NBUF + b
                pltpu.make_async_copy(
                    z_hbm.at[rowv.at[j]], bufs[b], gsems[b]).wait()
                sdescs.append(pltpu.async_copy(
                    bufs[b], acc.at[colv.at[j]], ssems[b], add=True))
            for b in range(NBUF):
                sdescs[b].wait()
                pltpu.async_copy(z_hbm.at[rowv.at[(r + 1) * NBUF + b]],
                                 bufs[b], gsems[b])
            return carry

        lax.fori_loop(0, CPS // NBUF - 1, round_fn, 0)
        for b in range(NBUF):
            j = CPS - NBUF + b
            pltpu.make_async_copy(
                z_hbm.at[rowv.at[j]], bufs[b], gsems[b]).wait()
            pltpu.sync_copy(bufs[b], acc.at[colv.at[j]], add=True)
    plsc.subcore_barrier()
    pltpu.sync_copy(acc.at[pl.ds(s * RPT, RPT)],
                    out_hbm.at[pl.ds(c * NP + s * RPT, RPT)])


_agg128 = pl.kernel(
    _agg_body,
    out_type=jax.ShapeDtypeStruct((2 * NP, D1), jnp.float32),
    mesh=_mesh,
    scratch_types=(
        [pltpu.VMEM((CPS, CH), jnp.int32),
         pltpu.VMEM((CPS, CH), jnp.int32)]
        + [pltpu.VMEM((CH, D1), jnp.float32) for _ in range(NBUF)]
        + [pltpu.VMEM_SHARED((NP, D1), jnp.float32)]
        + [pltpu.SemaphoreType.DMA for _ in range(2 * NBUF)]
    ),
)


def _deg_body(cols_hbm, zeros_hbm, ones_hbm, out_hbm, colv, ones_v, acc,
              ssem0, ssem1, ssem2, ssem3):
    ssems = (ssem0, ssem1, ssem2, ssem3)
    c = lax.axis_index("c")
    s = lax.axis_index("s")
    w = (c * NS + s) * SUB
    pltpu.sync_copy(ones_hbm, ones_v)
    pltpu.sync_copy(zeros_hbm, acc.at[pl.ds(s * RPT, RPT)])
    plsc.subcore_barrier()

    for u in range(SUB):
        pltpu.sync_copy(cols_hbm.at[w + u], colv)

        def round_fn(r, carry):
            descs = []
            for b in range(NBUF):
                descs.append(pltpu.async_copy(
                    ones_v, acc.at[colv.at[r * NBUF + b]], ssems[b],
                    add=True))
            for d in descs:
                d.wait()
            return carry

        lax.fori_loop(0, CPS // NBUF, round_fn, 0)
    plsc.subcore_barrier()
    pltpu.sync_copy(acc.at[pl.ds(s * RPT, RPT)],
                    out_hbm.at[pl.ds(c * NP + s * RPT, RPT)])


_deg = pl.kernel(
    _deg_body,
    out_type=jax.ShapeDtypeStruct((2 * NP, D1), jnp.float32),
    mesh=_mesh,
    scratch_types=[
        pltpu.VMEM((CPS, CH), jnp.int32),
        pltpu.VMEM((CH, D1), jnp.float32),
        pltpu.VMEM_SHARED((NP, D1), jnp.float32),
        pltpu.SemaphoreType.DMA,
        pltpu.SemaphoreType.DMA,
        pltpu.SemaphoreType.DMA,
        pltpu.SemaphoreType.DMA,
    ],
)


def _kw_body(a_ref, w2_ref, out_ref):
    out_ref[...] = jnp.dot(a_ref[...], w2_ref[...],
                           preferred_element_type=jnp.float32)


def _k1_body(cnt_ref, msk_ref, x_ref, dinv_ref, z0_ref):
    dinv = msk_ref[...] * lax.rsqrt(1.0 + cnt_ref[...][:, :1])
    dinvb = jnp.broadcast_to(dinv, (RB, D1))
    dinv_ref[...] = dinvb
    z0_ref[...] = dinvb * x_ref[...]


def _k2_body(acc_ref, z0_ref, dinv_ref, w12_ref, b12_ref, z2_ref):
    dinv = dinv_ref[...]
    a1 = dinv * (acc_ref[...] + z0_ref[...])
    h2 = jnp.dot(a1, w12_ref[...], preferred_element_type=jnp.float32)
    z2_ref[...] = dinv * (h2 + b12_ref[...])


def _k3_body(acc_ref, z2_ref, dinv_ref, w3_ref, b2_ref, z3_ref):
    dinv = dinv_ref[...]
    x2 = dinv * (acc_ref[...] + z2_ref[...]) + b2_ref[...]
    h3 = jnp.dot(x2, w3_ref[...], preferred_element_type=jnp.float32)
    z3 = dinv[:, :D3] * h3
    z3_ref[...] = jnp.concatenate(
        [z3, jnp.zeros((RB, D1 - D3), jnp.float32)], axis=1)


def _k4_body(acc_ref, z3_ref, dinv_ref, b3_ref, wl_ref, bl_ref,
             x3_ref, p_ref):
    x3 = (dinv_ref[...][:, :D3] * (acc_ref[...][:, :D3] + z3_ref[...][:, :D3])
          + b3_ref[...])
    x3_ref[...] = x3
    logits = jnp.dot(x3, wl_ref[...],
                     preferred_element_type=jnp.float32) + bl_ref[...]
    p_ref[...] = jax.nn.sigmoid(logits)


def _row_spec(w):
    return pl.BlockSpec((RB, w), lambda i: (i, 0))


def _full_spec(shape):
    return pl.BlockSpec(shape, lambda i: tuple(0 for _ in shape))


_GRID = (2 * NP // RB,)

_kw = pl.pallas_call(
    _kw_body,
    grid=(1,),
    in_specs=[_full_spec((136, 1000)), _full_spec((1000, D1))],
    out_specs=_full_spec((136, D1)),
    out_shape=jax.ShapeDtypeStruct((136, D1), jnp.float32),
)

_k1 = pl.pallas_call(
    _k1_body,
    grid=_GRID,
    in_specs=[_row_spec(D1), _row_spec(1), _row_spec(D1)],
    out_specs=[_row_spec(D1), _row_spec(D1)],
    out_shape=[jax.ShapeDtypeStruct((2 * NP, D1), jnp.float32),
               jax.ShapeDtypeStruct((2 * NP, D1), jnp.float32)],
)

_k2 = pl.pallas_call(
    _k2_body,
    grid=_GRID,
    in_specs=[_row_spec(D1), _row_spec(D1), _row_spec(D1),
              _full_spec((D1, D1)), _full_spec((1, D1))],
    out_specs=_row_spec(D1),
    out_shape=jax.ShapeDtypeStruct((2 * NP, D1), jnp.float32),
)

_k3 = pl.pallas_call(
    _k3_body,
    grid=_GRID,
    in_specs=[_row_spec(D1), _row_spec(D1), _row_spec(D1),
              _full_spec((D1, D3)), _full_spec((1, D1))],
    out_specs=_row_spec(D1),
    out_shape=jax.ShapeDtypeStruct((2 * NP, D1), jnp.float32),
)

_k4 = pl.pallas_call(
    _k4_body,
    grid=_GRID,
    in_specs=[_row_spec(D1), _row_spec(D1), _row_spec(D1),
              _full_spec((1, D3)), _full_spec((D3, D3)), _full_spec((1, D3))],
    out_specs=[_row_spec(D3), _row_spec(D3)],
    out_shape=[jax.ShapeDtypeStruct((2 * NP, D3), jnp.float32),
               jax.ShapeDtypeStruct((2 * NP, D3), jnp.float32)],
)


def _prep_edges(ei, g):
    pad = jnp.full((EP - E,), N, jnp.int32)
    rl = jnp.concatenate([ei[0], pad])
    c = jnp.concatenate([ei[1], pad])
    rg = rl + jnp.int32(g * NP)
    return rg.reshape(NS * SUB, CPS, CH), c.reshape(NS * SUB, CPS, CH)


def kernel(x_s, edge_index_s, x_t, edge_index_t,
           W1, b1, W2, b2, W3, b3, Wl, bl):
    f32 = jnp.float32
    rs, cs = _prep_edges(edge_index_s, 0)
    rt, ct = _prep_edges(edge_index_t, 1)
    rows = jnp.concatenate([rs, rt], axis=0)
    cols = jnp.concatenate([cs, ct], axis=0)

    padrows = jnp.zeros((NP - N, x_s.shape[1]), f32)
    x2g = jnp.concatenate([x_s, padrows, x_t, padrows], axis=0)

    mask = jnp.concatenate([jnp.ones((N, 1), f32), jnp.zeros((NP - N, 1), f32)])
    mask = jnp.concatenate([mask, mask], axis=0)

    a_in = jnp.concatenate([W1, b1[None, :], jnp.zeros((7, 1000), f32)])
    w2p = jnp.pad(W2, ((0, 0), (0, D1 - 100)))
    wb12 = _kw(a_in, w2p)
    w12, b12 = wb12[:128], wb12[128:129]
    w3p = jnp.pad(W3, ((0, D1 - 100), (0, 0)))
    b2p = jnp.pad(b2, (0, D1 - 100))[None, :]
    wlp = jnp.pad(Wl, ((0, 0), (0, D3 - 10)))
    blp = jnp.pad(bl, (0, D3 - 10))[None, :]

    z128 = jnp.zeros((RPT, D1), f32)

    cnt = _deg(cols, z128, jnp.ones((CH, D1), f32))
    dinvb, z0 = _k1(cnt, mask, x2g)
    acc1 = _agg128(z0, rows, cols, z128)
    z2 = _k2(acc1, z0, dinvb, w12, b12)
    acc2 = _agg128(z2, rows, cols, z128)
    z3 = _k3(acc2, z2, dinvb, w3p, b2p)
    acc3 = _agg128(z3, rows, cols, z128)
    x3, p = _k4(acc3, z3, dinvb, b3[None, :], wlp, blp)

    x_s_out = x3[:N]
    x_t_out = x3[NP:NP + N]
    pred = p[:N, :10]
    return (x_s_out, x_t_out, pred, pred)


# speedup vs baseline: 13.8092x; 1.1221x over previous
import jax
import jax.numpy as jnp
from jax import lax
from jax.experimental import pallas as pl
from jax.experimental.pallas import tpu as pltpu
from jax.experimental.pallas import tpu_sc as plsc

N = 10000
E = 320000
NP = 10240
NS = 16
CH = 128
NBUF = 2
CPT = 160
SUB = 8
CPS = CPT // SUB
EPT = CPT * CH
EP = NS * EPT
RPT = NP // NS
D1 = 128
D3 = 16
RB = 1024

_mesh = plsc.VectorSubcoreMesh(core_axis_name="c", subcore_axis_name="s")


def _agg_body(z_hbm, rows_hbm, cols_hbm, zeros_hbm, out_hbm, *scr):
    rowv, colv = scr[0], scr[1]
    bufs = scr[2:2 + NBUF]
    acc = scr[2 + NBUF]
    gsems = scr[3 + NBUF:3 + 2 * NBUF]
    ssems = scr[3 + 2 * NBUF:3 + 3 * NBUF]
    c = lax.axis_index("c")
    s = lax.axis_index("s")
    w = (c * NS + s) * SUB
    pltpu.sync_copy(zeros_hbm, acc.at[pl.ds(s * RPT, RPT)])
    plsc.subcore_barrier()

    for u in range(SUB):
        pltpu.sync_copy(rows_hbm.at[w + u], rowv)
        pltpu.sync_copy(cols_hbm.at[w + u], colv)
        for b in range(NBUF):
            pltpu.async_copy(z_hbm.at[rowv.at[b]], bufs[b], gsems[b])

        def round_fn(r, carry):
            sdescs = []
            for b in range(NBUF):
                j = r * NBUF + b
                pltpu.make_async_copy(
                    z_hbm.at[rowv.at[j]], bufs[b], gsems[b]).wait()
                sdescs.append(pltpu.async_copy(
                    bufs[b], acc.at[colv.at[j]], ssems[b], add=True))
            for b in range(NBUF):
                sdescs[b].wait()
                pltpu.async_copy(z_hbm.at[rowv.at[(r + 1) * NBUF + b]],
                                 bufs[b], gsems[b])
            return carry

        lax.fori_loop(0, CPS // NBUF - 1, round_fn, 0)
        for b in range(NBUF):
            j = CPS - NBUF + b
            pltpu.make_async_copy(
                z_hbm.at[rowv.at[j]], bufs[b], gsems[b]).wait()
            pltpu.sync_copy(bufs[b], acc.at[colv.at[j]], add=True)
    plsc.subcore_barrier()
    pltpu.sync_copy(acc.at[pl.ds(s * RPT, RPT)],
                    out_hbm.at[pl.ds(c * NP + s * RPT, RPT)])


_agg128 = pl.kernel(
    _agg_body,
    out_type=jax.ShapeDtypeStruct((2 * NP, D1), jnp.float32),
    mesh=_mesh,
    scratch_types=(
        [pltpu.VMEM((CPS, CH), jnp.int32),
         pltpu.VMEM((CPS, CH), jnp.int32)]
        + [pltpu.VMEM((CH, D1), jnp.float32) for _ in range(NBUF)]
        + [pltpu.VMEM_SHARED((NP, D1), jnp.float32)]
        + [pltpu.SemaphoreType.DMA for _ in range(2 * NBUF)]
    ),
)


def _deg_body(cols_hbm, zeros_hbm, ones_hbm, out_hbm, *scr):
    colv, ones_v, acc = scr[0], scr[1], scr[2]
    ssems = scr[3:3 + NBUF]
    c = lax.axis_index("c")
    s = lax.axis_index("s")
    w = (c * NS + s) * SUB
    pltpu.sync_copy(ones_hbm, ones_v)
    pltpu.sync_copy(zeros_hbm, acc.at[pl.ds(s * RPT, RPT)])
    plsc.subcore_barrier()

    for u in range(SUB):
        pltpu.sync_copy(cols_hbm.at[w + u], colv)

        def round_fn(r, carry):
            descs = []
            for b in range(NBUF):
                descs.append(pltpu.async_copy(
                    ones_v, acc.at[colv.at[r * NBUF + b]], ssems[b],
                    add=True))
            for d in descs:
                d.wait()
            return carry

        lax.fori_loop(0, CPS // NBUF, round_fn, 0)
    plsc.subcore_barrier()
    pltpu.sync_copy(acc.at[pl.ds(s * RPT, RPT)],
                    out_hbm.at[pl.ds(c * NP + s * RPT, RPT)])


_deg = pl.kernel(
    _deg_body,
    out_type=jax.ShapeDtypeStruct((2 * NP, D1), jnp.float32),
    mesh=_mesh,
    scratch_types=(
        [pltpu.VMEM((CPS, CH), jnp.int32),
         pltpu.VMEM((CH, D1), jnp.float32),
         pltpu.VMEM_SHARED((NP, D1), jnp.float32)]
        + [pltpu.SemaphoreType.DMA for _ in range(NBUF)]
    ),
)


def _kw_body(a_ref, w2_ref, out_ref):
    out_ref[...] = jnp.dot(a_ref[...], w2_ref[...],
                           preferred_element_type=jnp.float32)


def _k1_body(cnt_ref, msk_ref, x_ref, dinv_ref, z0_ref):
    dinv = msk_ref[...] * lax.rsqrt(1.0 + cnt_ref[...][:, :1])
    dinvb = jnp.broadcast_to(dinv, (RB, D1))
    dinv_ref[...] = dinvb
    z0_ref[...] = dinvb * x_ref[...]


def _k2_body(acc_ref, z0_ref, dinv_ref, w12_ref, b12_ref, z2_ref):
    dinv = dinv_ref[...]
    a1 = dinv * (acc_ref[...] + z0_ref[...])
    h2 = jnp.dot(a1, w12_ref[...], preferred_element_type=jnp.float32)
    z2_ref[...] = dinv * (h2 + b12_ref[...])


def _k3_body(acc_ref, z2_ref, dinv_ref, w3_ref, b2_ref, z3_ref):
    dinv = dinv_ref[...]
    x2 = dinv * (acc_ref[...] + z2_ref[...]) + b2_ref[...]
    h3 = jnp.dot(x2, w3_ref[...], preferred_element_type=jnp.float32)
    z3 = dinv[:, :D3] * h3
    z3_ref[...] = jnp.concatenate(
        [z3, jnp.zeros((RB, D1 - D3), jnp.float32)], axis=1)


def _k4_body(acc_ref, z3_ref, dinv_ref, b3_ref, wl_ref, bl_ref,
             x3_ref, p_ref):
    x3 = (dinv_ref[...][:, :D3] * (acc_ref[...][:, :D3] + z3_ref[...][:, :D3])
          + b3_ref[...])
    x3_ref[...] = x3
    logits = jnp.dot(x3, wl_ref[...],
                     preferred_element_type=jnp.float32) + bl_ref[...]
    p_ref[...] = jax.nn.sigmoid(logits)


def _row_spec(w):
    return pl.BlockSpec((RB, w), lambda i: (i, 0))


def _full_spec(shape):
    return pl.BlockSpec(shape, lambda i: tuple(0 for _ in shape))


_GRID = (2 * NP // RB,)

_kw = pl.pallas_call(
    _kw_body,
    grid=(1,),
    in_specs=[_full_spec((136, 1000)), _full_spec((1000, D1))],
    out_specs=_full_spec((136, D1)),
    out_shape=jax.ShapeDtypeStruct((136, D1), jnp.float32),
)

_k1 = pl.pallas_call(
    _k1_body,
    grid=_GRID,
    in_specs=[_row_spec(D1), _row_spec(1), _row_spec(D1)],
    out_specs=[_row_spec(D1), _row_spec(D1)],
    out_shape=[jax.ShapeDtypeStruct((2 * NP, D1), jnp.float32),
               jax.ShapeDtypeStruct((2 * NP, D1), jnp.float32)],
)

_k2 = pl.pallas_call(
    _k2_body,
    grid=_GRID,
    in_specs=[_row_spec(D1), _row_spec(D1), _row_spec(D1),
              _full_spec((D1, D1)), _full_spec((1, D1))],
    out_specs=_row_spec(D1),
    out_shape=jax.ShapeDtypeStruct((2 * NP, D1), jnp.float32),
)

_k3 = pl.pallas_call(
    _k3_body,
    grid=_GRID,
    in_specs=[_row_spec(D1), _row_spec(D1), _row_spec(D1),
              _full_spec((D1, D3)), _full_spec((1, D1))],
    out_specs=_row_spec(D1),
    out_shape=jax.ShapeDtypeStruct((2 * NP, D1), jnp.float32),
)

_k4 = pl.pallas_call(
    _k4_body,
    grid=_GRID,
    in_specs=[_row_spec(D1), _row_spec(D1), _row_spec(D1),
              _full_spec((1, D3)), _full_spec((D3, D3)), _full_spec((1, D3))],
    out_specs=[_row_spec(D3), _row_spec(D3)],
    out_shape=[jax.ShapeDtypeStruct((2 * NP, D3), jnp.float32),
               jax.ShapeDtypeStruct((2 * NP, D3), jnp.float32)],
)


def _prep_edges(ei, g):
    pad = jnp.full((EP - E,), N, jnp.int32)
    rl = jnp.concatenate([ei[0], pad])
    c = jnp.concatenate([ei[1], pad])
    rg = rl + jnp.int32(g * NP)
    return rg.reshape(NS * SUB, CPS, CH), c.reshape(NS * SUB, CPS, CH)


def kernel(x_s, edge_index_s, x_t, edge_index_t,
           W1, b1, W2, b2, W3, b3, Wl, bl):
    f32 = jnp.float32
    rs, cs = _prep_edges(edge_index_s, 0)
    rt, ct = _prep_edges(edge_index_t, 1)
    rows = jnp.concatenate([rs, rt], axis=0)
    cols = jnp.concatenate([cs, ct], axis=0)

    padrows = jnp.zeros((NP - N, x_s.shape[1]), f32)
    x2g = jnp.concatenate([x_s, padrows, x_t, padrows], axis=0)

    mask = jnp.concatenate([jnp.ones((N, 1), f32), jnp.zeros((NP - N, 1), f32)])
    mask = jnp.concatenate([mask, mask], axis=0)

    a_in = jnp.concatenate([W1, b1[None, :], jnp.zeros((7, 1000), f32)])
    w2p = jnp.pad(W2, ((0, 0), (0, D1 - 100)))
    wb12 = _kw(a_in, w2p)
    w12, b12 = wb12[:128], wb12[128:129]
    w3p = jnp.pad(W3, ((0, D1 - 100), (0, 0)))
    b2p = jnp.pad(b2, (0, D1 - 100))[None, :]
    wlp = jnp.pad(Wl, ((0, 0), (0, D3 - 10)))
    blp = jnp.pad(bl, (0, D3 - 10))[None, :]

    z128 = jnp.zeros((RPT, D1), f32)

    cnt = _deg(cols, z128, jnp.ones((CH, D1), f32))
    dinvb, z0 = _k1(cnt, mask, x2g)
    acc1 = _agg128(z0, rows, cols, z128)
    z2 = _k2(acc1, z0, dinvb, w12, b12)
    acc2 = _agg128(z2, rows, cols, z128)
    z3 = _k3(acc2, z2, dinvb, w3p, b2p)
    acc3 = _agg128(z3, rows, cols, z128)
    x3, p = _k4(acc3, z3, dinvb, b3[None, :], wlp, blp)

    x_s_out = x3[:N]
    x_t_out = x3[NP:NP + N]
    pred = p[:N, :10]
    return (x_s_out, x_t_out, pred, pred)
